# baseline (speedup 1.0000x reference)
import jax
import jax.numpy as jnp
from jax import lax
from jax.experimental import pallas as pl
from jax.experimental.pallas import tpu as pltpu
from jax.experimental.pallas import tpu_sc as plsc

T = 4
B = 128
VOCAB = 100000
ATTN = 200
OOV = 100
VEXT = VOCAB + OOV
ROWS = T * B
LANES = 16
NW = 32
ROWS_PER_W = ROWS // NW
BUF = 100112

IW = 112
SPLIT = 104

VB = 2048
NVB = (VEXT + VB - 1) // VB


def _tc_body(pg_ref, vd_ref, out_ref):
    j = pl.program_id(0)
    v = j * VB + lax.broadcasted_iota(jnp.int32, (VB, T, 128), 0)
    pg = pg_ref[...][None]
    vals = pg * jnp.transpose(vd_ref[...], (1, 0, 2))
    out_ref[...] = jnp.where(v < VOCAB, vals, 0.0)


def _sc_body(attn_hbm, pg_hbm, idx_hbm, out_ref,
             ivb, avb, ovb, buf, pgv, sem):
    wid = lax.axis_index("s") * 2 + lax.axis_index("c")
    lanes = lax.iota(jnp.int32, LANES)
    zf = jnp.zeros((LANES,), jnp.float32)
    zi = jnp.zeros((LANES,), jnp.int32)
    row0 = wid * ROWS_PER_W
    b0 = lax.rem(row0, B)
    RW = 2 * IW

    pltpu.sync_copy(pg_hbm, pgv.at[pl.ds(0, ROWS)])

    def stage(r, c):
        pltpu.async_copy(idx_hbm.at[pl.ds((b0 + r) * ATTN, SPLIT)],
                         ivb.at[pl.ds(r * RW, SPLIT)], sem)
        pltpu.async_copy(idx_hbm.at[pl.ds((b0 + r) * ATTN + SPLIT,
                                          ATTN - SPLIT)],
                         ivb.at[pl.ds(r * RW + IW, ATTN - SPLIT)], sem)
        pltpu.async_copy(attn_hbm.at[pl.ds((row0 + r) * ATTN, SPLIT)],
                         avb.at[pl.ds(r * RW, SPLIT)], sem)
        pltpu.async_copy(attn_hbm.at[pl.ds((row0 + r) * ATTN + SPLIT,
                                           ATTN - SPLIT)],
                         avb.at[pl.ds(r * RW + IW, ATTN - SPLIT)], sem)
        return c
    lax.fori_loop(0, ROWS_PER_W, stage, 0)

    def stage_drain(r, c):
        pltpu.make_async_copy(idx_hbm.at[pl.ds((b0 + r) * ATTN, SPLIT)],
                              ivb.at[pl.ds(r * RW, SPLIT)], sem).wait()
        pltpu.make_async_copy(idx_hbm.at[pl.ds((b0 + r) * ATTN + SPLIT,
                                               ATTN - SPLIT)],
                              ivb.at[pl.ds(r * RW + IW, ATTN - SPLIT)],
                              sem).wait()
        pltpu.make_async_copy(attn_hbm.at[pl.ds((row0 + r) * ATTN, SPLIT)],
                              avb.at[pl.ds(r * RW, SPLIT)], sem).wait()
        pltpu.make_async_copy(attn_hbm.at[pl.ds((row0 + r) * ATTN + SPLIT,
                                                ATTN - SPLIT)],
                              avb.at[pl.ds(r * RW + IW, ATTN - SPLIT)],
                              sem).wait()
        return c
    lax.fori_loop(0, ROWS_PER_W, stage_drain, 0)

    def convert(r, c):
        row = row0 + r
        o0 = r * RW
        ivb[pl.ds(o0 + 96, LANES)] = jnp.where(
            lanes < SPLIT - 96, ivb[pl.ds(o0 + 96, LANES)], zi)
        ivb[pl.ds(o0 + IW + 96, LANES)] = zi
        avb[pl.ds(o0 + 96, LANES)] = jnp.where(
            lanes < SPLIT - 96, avb[pl.ds(o0 + 96, LANES)], zf)
        avb[pl.ds(o0 + IW + 96, LANES)] = zf
        for cc in range(RW // LANES):
            o = o0 + cc * LANES
            ivb[pl.ds(o, LANES)] = ivb[pl.ds(o, LANES)] * 512 + row
        return c
    lax.fori_loop(0, ROWS_PER_W, convert, 0)

    def fire_gather(r, c):
        for si in range(2):
            o = r * RW + si * IW
            pltpu.async_copy(out_ref.at[ivb.at[pl.ds(o, IW)]],
                             ovb.at[pl.ds(o, IW)], sem)
        return c
    lax.fori_loop(0, ROWS_PER_W, fire_gather, 0)

    def process(r, c):
        row = row0 + r
        o0 = r * RW
        for si in range(2):
            o = o0 + si * IW
            pltpu.make_async_copy(out_ref.at[ivb.at[pl.ds(o, IW)]],
                                  ovb.at[pl.ds(o, IW)], sem).wait()
        pgwin = pgv[pl.ds(row, LANES)]
        omg = jnp.ones((LANES,), jnp.float32) - (zf + pgwin[0])
        for cc in range(RW // LANES):
            o = o0 + cc * LANES
            vloc = lax.shift_right_logical(ivb[pl.ds(o, LANES)], 9)
            plsc.store_scatter(buf, [vloc], ovb[pl.ds(o, LANES)])
        for cc in range(RW // LANES):
            o = o0 + cc * LANES
            vloc = lax.shift_right_logical(ivb[pl.ds(o, LANES)], 9)
            vals = avb[pl.ds(o, LANES)] * omg
            for lane in range(LANES):
                plsc.addupdate_scatter(buf, [vloc], vals,
                                       mask=lanes == lane)
        for cc in range(RW // LANES):
            o = o0 + cc * LANES
            vloc = lax.shift_right_logical(ivb[pl.ds(o, LANES)], 9)
            ovb[pl.ds(o, LANES)] = plsc.load_gather(buf, [vloc])
        for si in range(2):
            o = o0 + si * IW
            pltpu.async_copy(ovb.at[pl.ds(o, IW)],
                             out_ref.at[ivb.at[pl.ds(o, IW)]], sem)
        return c
    lax.fori_loop(0, ROWS_PER_W, process, 0)

    def drain_out(r, c):
        for si in range(2):
            o = r * RW + si * IW
            pltpu.make_async_copy(ovb.at[pl.ds(o, IW)],
                                  out_ref.at[ivb.at[pl.ds(o, IW)]],
                                  sem).wait()
        return c
    lax.fori_loop(0, ROWS_PER_W, drain_out, 0)


@jax.jit
def _final_dist(vocab_dists, attn_dists, p_gens, enc_batch_extend_vocab):
    vocab_t = jnp.transpose(vocab_dists, (0, 2, 1))
    pg2 = p_gens.reshape(T, B)

    dense = pl.pallas_call(
        _tc_body,
        grid=(NVB,),
        in_specs=[
            pl.BlockSpec((T, B), lambda j: (0, 0)),
            pl.BlockSpec((T, VB, B), lambda j: (0, j, 0)),
        ],
        out_specs=pl.BlockSpec((VB, T, B), lambda j: (j, 0, 0)),
        out_shape=jax.ShapeDtypeStruct((VEXT, T, B), jnp.float32),
        compiler_params=pltpu.CompilerParams(
            dimension_semantics=("arbitrary",)),
    )(pg2, vocab_t)

    attn1 = attn_dists.reshape(ROWS * ATTN)
    pg1 = p_gens.reshape(ROWS)
    idx1 = enc_batch_extend_vocab.reshape(B * ATTN)

    mesh = plsc.VectorSubcoreMesh(core_axis_name="c", subcore_axis_name="s")
    rmw = pl.kernel(
        _sc_body,
        out_type=(),
        mesh=mesh,
        compiler_params=pltpu.CompilerParams(needs_layout_passes=False),
        scratch_types=[
            pltpu.VMEM((ROWS_PER_W * 2 * IW,), jnp.int32),
            pltpu.VMEM((ROWS_PER_W * 2 * IW,), jnp.float32),
            pltpu.VMEM((ROWS_PER_W * 2 * IW,), jnp.float32),
            pltpu.VMEM((BUF,), jnp.float32),
            pltpu.VMEM((ROWS + LANES,), jnp.float32),
            pltpu.SemaphoreType.DMA,
        ],
    )
    ref = jax.new_ref(dense.reshape(VEXT * ROWS))
    rmw(attn1, pg1, idx1, ref)
    out_t = ref[...].reshape(VEXT, T, B)
    return jnp.transpose(out_t, (1, 2, 0))


def kernel(vocab_dists, attn_dists, p_gens, enc_batch_extend_vocab):
    return _final_dist(vocab_dists, attn_dists, p_gens,
                       enc_batch_extend_vocab)
